# baseline (speedup 1.0000x reference)
import jax
import jax.numpy as jnp
from jax import lax
from jax.experimental import pallas as pl
from jax.experimental.pallas import tpu as pltpu
from jax.experimental.pallas import tpu_sc as plsc

NC = 2
NS = 16
NW = NC * NS
GA = 128
GB = 80
NBUF = 2
L = 16
T = 200


def _body(idx_hbm, wcr_hbm, out_hbm, idx_v, iqa, iqb, vga0, vgb0, vga1,
          vgb1, vo, sem_g0, sem_g1, sem_s0, sem_s1):
    spw = out_hbm.shape[0] // NW
    quads = wcr_hbm.shape[0] // 4

    cid = lax.axis_index("c")
    sid = lax.axis_index("s")
    wid = sid * NC + cid
    vga = (vga0, vga1)
    vgb = (vgb0, vgb1)
    sem_g = (sem_g0, sem_g1)
    sem_s = (sem_s0, sem_s1)
    s0 = wid * spw

    lane = lax.iota(jnp.int32, L)
    pad_line = wid

    pltpu.sync_copy(idx_hbm.at[wid], idx_v.at[pl.ds(0, spw * T)])

    def _prep(s):
        for t in range(T // L):
            toks = idx_v[pl.ds(s * T + t * L, L)]
            lines = (toks & 3) * quads + lax.shift_right_logical(toks, 2)
            if t < 8:
                iqa[pl.ds(t * L, L)] = lines
            else:
                iqb[pl.ds((t - 8) * L, L)] = lines
        toks = idx_v[pl.ds(s * T + 192, L)]
        lines = (toks & 3) * quads + lax.shift_right_logical(toks, 2)
        iqb[pl.ds(64, L)] = jnp.where(lane < 8, lines, pad_line)

    def _fire_gathers(b):
        pltpu.async_copy(wcr_hbm.at[iqa], vga[b], sem_g[b])
        pltpu.async_copy(wcr_hbm.at[iqb], vgb[b], sem_g[b])

    def _wait_gathers(b):
        pltpu.make_async_copy(wcr_hbm.at[pl.ds(0, GA)], vga[b],
                              sem_g[b]).wait()
        pltpu.make_async_copy(wcr_hbm.at[pl.ds(0, GB)], vgb[b],
                              sem_g[b]).wait()

    def _extract(b):
        @pl.loop(0, GA // 4)
        def _(q):
            for r in range(4):
                j = q * 4 + r
                vo[b, j, pl.ds(0, L)] = vga[b][j, pl.ds(0, L)]
                vo[b, j, pl.ds(L, L)] = vga[b][j, pl.ds(L, L)]

        @pl.loop(0, (T - GA) // 4)
        def _(q):
            for r in range(4):
                j = q * 4 + r
                vo[b, GA + j, pl.ds(0, L)] = vgb[b][j, pl.ds(0, L)]
                vo[b, GA + j, pl.ds(L, L)] = vgb[b][j, pl.ds(L, L)]

    _prep(0)
    _fire_gathers(0)

    @pl.loop(0, spw // NBUF)
    def _grp(i):
        for b in range(NBUF):
            s = i * NBUF + b
            samp = s0 + s

            _wait_gathers(b)

            @pl.when(s + 1 < spw)
            def _():
                _prep(s + 1)
                _fire_gathers(1 - b)

            @pl.when(s >= NBUF)
            def _():
                pltpu.make_async_copy(vo.at[b], out_hbm.at[samp - NBUF],
                                      sem_s[b]).wait()

            _extract(b)
            pltpu.async_copy(vo.at[b], out_hbm.at[samp], sem_s[b])

    for b in range(NBUF):
        s = spw - NBUF + b
        pltpu.make_async_copy(vo.at[b], out_hbm.at[s0 + s], sem_s[b]).wait()


def kernel(token_ids, weight):
    v, d = weight.shape
    ns, t = token_ids.shape
    assert d == 32 and v % 4 == 0 and t == T and ns % NW == 0
    quads = v // 4
    spw = ns // NW

    ids = token_ids.astype(jnp.int32).reshape(NW, spw * T)
    flat = weight.reshape(-1)
    rots = [
        jnp.concatenate([flat[32 * r:],
                         jnp.zeros((32 * r,), jnp.float32)])
        .reshape(quads, 4 * d)
        for r in range(4)
    ]
    wcr = jnp.concatenate(rots, axis=0)

    k = pl.kernel(
        _body,
        out_type=jax.ShapeDtypeStruct((ns, t, d), jnp.float32),
        mesh=plsc.VectorSubcoreMesh(core_axis_name="c", subcore_axis_name="s"),
        compiler_params=pltpu.CompilerParams(use_tc_tiling_on_sc=True),
        scratch_types=[
            pltpu.VMEM((spw * T + L, ), jnp.int32),
            pltpu.VMEM((GA,), jnp.int32),
            pltpu.VMEM((GB,), jnp.int32),
            pltpu.VMEM((GA, 4 * d), jnp.float32),
            pltpu.VMEM((GB, 4 * d), jnp.float32),
            pltpu.VMEM((GA, 4 * d), jnp.float32),
            pltpu.VMEM((GB, 4 * d), jnp.float32),
            pltpu.VMEM((NBUF, T, d), jnp.float32),
            pltpu.SemaphoreType.DMA,
            pltpu.SemaphoreType.DMA,
            pltpu.SemaphoreType.DMA,
            pltpu.SemaphoreType.DMA,
        ],
    )
    return k(ids, wcr)


# speedup vs baseline: 2.0835x; 2.0835x over previous
import jax
import jax.numpy as jnp
from jax import lax
from jax.experimental import pallas as pl
from jax.experimental.pallas import tpu as pltpu
from jax.experimental.pallas import tpu_sc as plsc

NC = 2
NS = 16
NW = NC * NS
GA = 128
GB = 80
NBUF = 2
L = 16
T = 200


def _body(idx_hbm, wcr_hbm, out_hbm, idx_v, iqa, iqb, vga0, vgb0, vga1,
          vgb1, vo, sem_g0, sem_g1, sem_s0, sem_s1):
    spw = out_hbm.shape[0] // NW

    cid = lax.axis_index("c")
    sid = lax.axis_index("s")
    wid = sid * NC + cid
    vga = (vga0, vga1)
    vgb = (vgb0, vgb1)
    sem_g = (sem_g0, sem_g1)
    sem_s = (sem_s0, sem_s1)
    s0 = wid * spw

    lane = lax.iota(jnp.int32, L)
    pad_line = wid

    pltpu.sync_copy(idx_hbm.at[wid], idx_v.at[pl.ds(0, spw * T)])

    def _prep(s):
        for t in range(T // L):
            lines = idx_v[pl.ds(s * T + t * L, L)]
            if t < 8:
                iqa[pl.ds(t * L, L)] = lines
            else:
                iqb[pl.ds((t - 8) * L, L)] = lines
        lines = idx_v[pl.ds(s * T + 192, L)]
        iqb[pl.ds(64, L)] = jnp.where(lane < 8, lines, pad_line)

    def _fire_gathers(b):
        pltpu.async_copy(wcr_hbm.at[iqa], vga[b], sem_g[b])
        pltpu.async_copy(wcr_hbm.at[iqb], vgb[b], sem_g[b])

    def _wait_gathers(b):
        pltpu.make_async_copy(wcr_hbm.at[pl.ds(0, GA)], vga[b],
                              sem_g[b]).wait()
        pltpu.make_async_copy(wcr_hbm.at[pl.ds(0, GB)], vgb[b],
                              sem_g[b]).wait()

    def _extract(b):
        @pl.loop(0, GA // 4)
        def _(q):
            for r in range(4):
                j = q * 4 + r
                vo[b, j, pl.ds(0, L)] = vga[b][j, pl.ds(0, L)]
                vo[b, j, pl.ds(L, L)] = vga[b][j, pl.ds(L, L)]

        @pl.loop(0, (T - GA) // 4)
        def _(q):
            for r in range(4):
                j = q * 4 + r
                vo[b, GA + j, pl.ds(0, L)] = vgb[b][j, pl.ds(0, L)]
                vo[b, GA + j, pl.ds(L, L)] = vgb[b][j, pl.ds(L, L)]

    _prep(0)
    _fire_gathers(0)

    @pl.loop(0, spw // NBUF)
    def _grp(i):
        for b in range(NBUF):
            s = i * NBUF + b
            samp = s0 + s

            _wait_gathers(b)

            @pl.when(s + 1 < spw)
            def _():
                _prep(s + 1)
                _fire_gathers(1 - b)

            @pl.when(s >= NBUF)
            def _():
                pltpu.make_async_copy(vo.at[b], out_hbm.at[samp - NBUF],
                                      sem_s[b]).wait()

            _extract(b)
            pltpu.async_copy(vo.at[b], out_hbm.at[samp], sem_s[b])

    for b in range(NBUF):
        s = spw - NBUF + b
        pltpu.make_async_copy(vo.at[b], out_hbm.at[s0 + s], sem_s[b]).wait()


def kernel(token_ids, weight):
    v, d = weight.shape
    ns, t = token_ids.shape
    assert d == 32 and v % 4 == 0 and t == T and ns % NW == 0
    spw = ns // NW

    ids = token_ids.astype(jnp.int32).reshape(NW, spw * T)
    wcr = jnp.pad(weight, ((0, 0), (0, 3 * d)))

    k = pl.kernel(
        _body,
        out_type=jax.ShapeDtypeStruct((ns, t, d), jnp.float32),
        mesh=plsc.VectorSubcoreMesh(core_axis_name="c", subcore_axis_name="s"),
        compiler_params=pltpu.CompilerParams(use_tc_tiling_on_sc=True),
        scratch_types=[
            pltpu.VMEM((spw * T + L, ), jnp.int32),
            pltpu.VMEM((GA,), jnp.int32),
            pltpu.VMEM((GB,), jnp.int32),
            pltpu.VMEM((GA, 4 * d), jnp.float32),
            pltpu.VMEM((GB, 4 * d), jnp.float32),
            pltpu.VMEM((GA, 4 * d), jnp.float32),
            pltpu.VMEM((GB, 4 * d), jnp.float32),
            pltpu.VMEM((NBUF, T, d), jnp.float32),
            pltpu.SemaphoreType.DMA,
            pltpu.SemaphoreType.DMA,
            pltpu.SemaphoreType.DMA,
            pltpu.SemaphoreType.DMA,
        ],
    )
    return k(ids, wcr)


# speedup vs baseline: 2.4200x; 1.1615x over previous
import jax
import jax.numpy as jnp
from jax import lax
from jax.experimental import pallas as pl
from jax.experimental.pallas import tpu as pltpu
from jax.experimental.pallas import tpu_sc as plsc

NC = 2
NS = 16
NW = NC * NS
GA = 128
G = 10
NBUF = 2


def _body(idx_hbm, table_hbm, out_hbm, idx_v, rows_v, sem_g, sem_s0, sem_s1):
    ng = idx_hbm.shape[1]
    wid = lax.axis_index("s") * NC + lax.axis_index("c")
    pltpu.sync_copy(idx_hbm.at[wid], idx_v)
    sem_s = (sem_s0, sem_s1)

    @pl.loop(0, ng // NBUF)
    def _outer(i):
        for b in range(NBUF):
            g = i * NBUF + b

            @pl.when(g >= NBUF)
            def _():
                pltpu.make_async_copy(
                    rows_v.at[b], out_hbm.at[wid, g - NBUF], sem_s[b]
                ).wait()

            for j in range(G):
                pltpu.async_copy(
                    table_hbm.at[idx_v.at[g, j]], rows_v.at[b, j], sem_g
                )
            pltpu.make_async_copy(out_hbm.at[wid, g], rows_v.at[b], sem_g).wait()
            pltpu.async_copy(rows_v.at[b], out_hbm.at[wid, g], sem_s[b])

    for b in range(NBUF):
        g = ng - NBUF + b
        pltpu.make_async_copy(rows_v.at[b], out_hbm.at[wid, g], sem_s[b]).wait()


def kernel(token_ids, weight):
    d = weight.shape[1]
    total = 1
    for s in token_ids.shape:
        total *= s
    per_w = total // NW
    ng = per_w // (G * GA)
    assert total == NW * ng * G * GA and ng % NBUF == 0

    ids = token_ids.reshape(-1).astype(jnp.int32).reshape(NW, ng, G, GA)

    k = pl.kernel(
        _body,
        out_type=jax.ShapeDtypeStruct((NW, ng, G, GA, d), jnp.float32),
        mesh=plsc.VectorSubcoreMesh(core_axis_name="c", subcore_axis_name="s"),
        compiler_params=pltpu.CompilerParams(use_tc_tiling_on_sc=False),
        scratch_types=[
            pltpu.VMEM((ng, G, GA), jnp.int32),
            pltpu.VMEM((NBUF, G, GA, d), jnp.float32),
            pltpu.SemaphoreType.DMA,
            pltpu.SemaphoreType.DMA,
            pltpu.SemaphoreType.DMA,
        ],
    )
    out = k(ids, weight)
    return out.reshape(*token_ids.shape, d)
